# Initial kernel scaffold; baseline (speedup 1.0000x reference)
import jax
import jax.numpy as jnp
from jax.experimental import pallas as pl


def kernel(edge_attr, W0, W1, W2):
    raise NotImplementedError("write your pallas kernel here")


# speedup vs baseline: 1.0939x; 1.0939x over previous
import functools

import jax
import jax.numpy as jnp
from jax import lax
from jax.experimental import pallas as pl
from jax.experimental.pallas import tpu as pltpu
from jax.experimental.pallas import tpu_sc as plsc

_NC = 2
_NS = 16
_NW = _NC * _NS
_LANES = 16


def _pick_chunk(per_w: int) -> int:
    for c in range(min(per_w, 512), 7, -1):
        if c % 8 == 0 and per_w % c == 0:
            return c
    return 0


@functools.partial(jax.jit, static_argnames=("n1", "n2"))
def _sc_combo_gather(a0, a1, a2, combo, *, n1, n2):
    e = a0.shape[0]
    d = combo.shape[1]
    per_w = e // _NW
    chunk = _pick_chunk(per_w)
    assert per_w * _NW == e and chunk, f"unsupported edge count {e}"
    nchunk = per_w // chunk
    m0 = n1 * n2

    mesh = plsc.VectorSubcoreMesh(core_axis_name="c", subcore_axis_name="s")

    @functools.partial(
        pl.kernel,
        mesh=mesh,
        out_type=jax.ShapeDtypeStruct((e, d), jnp.float32),
        scratch_types=[
            pltpu.VMEM((chunk,), jnp.int32),
            pltpu.VMEM((chunk,), jnp.int32),
            pltpu.VMEM((chunk,), jnp.int32),
            pltpu.VMEM((chunk,), jnp.int32),
            pltpu.VMEM((chunk, d), jnp.float32),
            pltpu.SemaphoreType.DMA,
        ],
    )
    def k(a0_hbm, a1_hbm, a2_hbm, combo_hbm, out_hbm,
          a0_v, a1_v, a2_v, idx_v, rows_v, sem):
        wid = lax.axis_index("s") * _NC + lax.axis_index("c")
        base = wid * per_w

        def chunk_body(g, carry):
            off = base + g * chunk
            pltpu.sync_copy(a0_hbm.at[pl.ds(off, chunk)], a0_v)
            pltpu.sync_copy(a1_hbm.at[pl.ds(off, chunk)], a1_v)
            pltpu.sync_copy(a2_hbm.at[pl.ds(off, chunk)], a2_v)

            def fuse(i, c):
                s = pl.ds(i * _LANES, _LANES)
                idx_v[s] = a0_v[s] * m0 + a1_v[s] * n2 + a2_v[s]
                return c

            lax.fori_loop(0, chunk // _LANES, fuse, 0, unroll=True)
            pltpu.async_copy(combo_hbm.at[idx_v], rows_v, sem).wait()
            pltpu.sync_copy(rows_v, out_hbm.at[pl.ds(off, chunk)])
            return carry

        lax.fori_loop(0, nchunk, chunk_body, 0)

    return k(a0, a1, a2, combo)


def kernel(edge_attr, W0, W1, W2):
    ea = edge_attr.astype(jnp.int32)
    n1, n2 = W1.shape[0], W2.shape[0]
    combo = (W0[:, None, None, :] + W1[None, :, None, :]
             + W2[None, None, :, :]).reshape(-1, W0.shape[1])
    return _sc_combo_gather(ea[:, 0], ea[:, 1], ea[:, 2], combo,
                            n1=n1, n2=n2)


# speedup vs baseline: 4.2425x; 3.8783x over previous
import functools

import jax
import jax.numpy as jnp
from jax import lax
from jax.experimental import pallas as pl
from jax.experimental.pallas import tpu as pltpu
from jax.experimental.pallas import tpu_sc as plsc

_NC = 2
_NS = 16
_NW = _NC * _NS
_LANES = 16


def _pick_chunk(per_w: int) -> int:
    for c in range(min(per_w, 512), 7, -1):
        if c % 8 == 0 and per_w % c == 0:
            return c
    return 0


@functools.partial(jax.jit, static_argnames=("n1", "n2"))
def _sc_combo_gather(a0, a1, a2, combo, *, n1, n2):
    e = a0.shape[0]
    d = combo.shape[1]
    per_w = e // _NW
    chunk = _pick_chunk(per_w)
    assert per_w * _NW == e and chunk, f"unsupported edge count {e}"
    nchunk = per_w // chunk
    m0 = n1 * n2
    n_combo = combo.shape[0] // _NW

    mesh = plsc.VectorSubcoreMesh(core_axis_name="c", subcore_axis_name="s")

    @functools.partial(
        pl.kernel,
        mesh=mesh,
        out_type=jax.ShapeDtypeStruct((e, d), jnp.float32),
        scratch_types=[
            pltpu.VMEM((chunk,), jnp.int32),
            pltpu.VMEM((chunk,), jnp.int32),
            pltpu.VMEM((chunk,), jnp.int32),
            pltpu.VMEM((chunk,), jnp.int32),
            pltpu.VMEM((chunk, d), jnp.float32),
            pltpu.SemaphoreType.DMA,
        ],
    )
    def k(a0_hbm, a1_hbm, a2_hbm, combo_hbm, out_hbm,
          a0_v, a1_v, a2_v, idx_v, rows_v, sem):
        wid = lax.axis_index("s") * _NC + lax.axis_index("c")
        base = wid * per_w

        def chunk_body(g, carry):
            off = base + g * chunk
            pltpu.sync_copy(a0_hbm.at[pl.ds(off, chunk)], a0_v)
            pltpu.sync_copy(a1_hbm.at[pl.ds(off, chunk)], a1_v)
            pltpu.sync_copy(a2_hbm.at[pl.ds(off, chunk)], a2_v)

            rep_off = wid * n_combo

            def fuse(i, c):
                s = pl.ds(i * _LANES, _LANES)
                idx_v[s] = a0_v[s] * m0 + a1_v[s] * n2 + a2_v[s] + rep_off
                return c

            lax.fori_loop(0, chunk // _LANES, fuse, 0, unroll=True)
            pltpu.async_copy(combo_hbm.at[idx_v], rows_v, sem).wait()
            pltpu.sync_copy(rows_v, out_hbm.at[pl.ds(off, chunk)])
            return carry

        lax.fori_loop(0, nchunk, chunk_body, 0)

    return k(a0, a1, a2, combo)


def kernel(edge_attr, W0, W1, W2):
    ea = edge_attr.astype(jnp.int32)
    n1, n2 = W1.shape[0], W2.shape[0]
    combo = (W0[:, None, None, :] + W1[None, :, None, :]
             + W2[None, None, :, :]).reshape(-1, W0.shape[1])
    combo = jnp.tile(combo, (_NW, 1))
    return _sc_combo_gather(ea[:, 0], ea[:, 1], ea[:, 2], combo,
                            n1=n1, n2=n2)


# speedup vs baseline: 7.7014x; 1.8153x over previous
import functools

import jax
import jax.numpy as jnp
from jax import lax
from jax.experimental import pallas as pl
from jax.experimental.pallas import tpu as pltpu
from jax.experimental.pallas import tpu_sc as plsc

_NC = 2
_NS = 16
_NW = _NC * _NS
_LANES = 16
_KREP = 8


def _pick_chunk(per_w: int) -> int:
    for c in range(min(per_w, 512), 7, -1):
        if c % 8 == 0 and per_w % c == 0:
            return c
    return 0


@functools.partial(jax.jit, static_argnames=("n1", "n2"))
def _sc_combo_gather(a0, a1, a2, combo, *, n1, n2):
    e = a0.shape[0]
    d = combo.shape[1]
    per_w = e // _NW
    chunk = _pick_chunk(per_w)
    assert per_w * _NW == e and chunk, f"unsupported edge count {e}"
    nchunk = per_w // chunk
    m0 = n1 * n2
    n_combo = combo.shape[0] // (_NW * _KREP)

    mesh = plsc.VectorSubcoreMesh(core_axis_name="c", subcore_axis_name="s")

    @functools.partial(
        pl.kernel,
        mesh=mesh,
        out_type=jax.ShapeDtypeStruct((e, d), jnp.float32),
        scratch_types=[
            pltpu.VMEM((chunk,), jnp.int32),
            pltpu.VMEM((chunk,), jnp.int32),
            pltpu.VMEM((chunk,), jnp.int32),
            pltpu.VMEM((chunk,), jnp.int32),
            pltpu.VMEM((chunk, d), jnp.float32),
            pltpu.SemaphoreType.DMA,
        ],
    )
    def k(a0_hbm, a1_hbm, a2_hbm, combo_hbm, out_hbm,
          a0_v, a1_v, a2_v, idx_v, rows_v, sem):
        wid = lax.axis_index("s") * _NC + lax.axis_index("c")
        base = wid * per_w

        def chunk_body(g, carry):
            off = base + g * chunk
            pltpu.sync_copy(a0_hbm.at[pl.ds(off, chunk)], a0_v)
            pltpu.sync_copy(a1_hbm.at[pl.ds(off, chunk)], a1_v)
            pltpu.sync_copy(a2_hbm.at[pl.ds(off, chunk)], a2_v)

            rep_base = wid * _KREP * n_combo

            for i in range(chunk // _LANES):
                s = pl.ds(i * _LANES, _LANES)
                rep_off = rep_base + (i % _KREP) * n_combo
                idx_v[s] = a0_v[s] * m0 + a1_v[s] * n2 + a2_v[s] + rep_off
            pltpu.async_copy(combo_hbm.at[idx_v], rows_v, sem).wait()
            pltpu.sync_copy(rows_v, out_hbm.at[pl.ds(off, chunk)])
            return carry

        lax.fori_loop(0, nchunk, chunk_body, 0)

    return k(a0, a1, a2, combo)


def kernel(edge_attr, W0, W1, W2):
    ea = edge_attr.astype(jnp.int32)
    n1, n2 = W1.shape[0], W2.shape[0]
    combo = (W0[:, None, None, :] + W1[None, :, None, :]
             + W2[None, None, :, :]).reshape(-1, W0.shape[1])
    combo = jnp.tile(combo, (_NW * _KREP, 1))
    return _sc_combo_gather(ea[:, 0], ea[:, 1], ea[:, 2], combo,
                            n1=n1, n2=n2)


# speedup vs baseline: 8.0146x; 1.0407x over previous
import functools

import jax
import jax.numpy as jnp
from jax import lax
from jax.experimental import pallas as pl
from jax.experimental.pallas import tpu as pltpu
from jax.experimental.pallas import tpu_sc as plsc

_NC = 2
_NS = 16
_NW = _NC * _NS
_LANES = 16
_KREP = 8


def _pick_chunk(per_w: int) -> int:
    for c in range(min(per_w, 400), 7, -1):
        if c % 8 == 0 and per_w % c == 0:
            return c
    return 0


@functools.partial(jax.jit, static_argnames=("n1", "n2"))
def _sc_combo_gather(a0, a1, a2, combo, *, n1, n2):
    e = a0.shape[0]
    d = combo.shape[1]
    per_w = e // _NW
    chunk = _pick_chunk(per_w)
    assert per_w * _NW == e and chunk, f"unsupported edge count {e}"
    nchunk = per_w // chunk
    m0 = n1 * n2
    n_combo = combo.shape[0] // (_NW * _KREP)

    mesh = plsc.VectorSubcoreMesh(core_axis_name="c", subcore_axis_name="s")

    @functools.partial(
        pl.kernel,
        mesh=mesh,
        out_type=jax.ShapeDtypeStruct((e, d), jnp.float32),
        scratch_types=[
            pltpu.VMEM((per_w,), jnp.int32),
            pltpu.VMEM((per_w,), jnp.int32),
            pltpu.VMEM((chunk, d), jnp.float32),
            pltpu.VMEM((chunk, d), jnp.float32),
            pltpu.SemaphoreType.DMA,
            pltpu.SemaphoreType.DMA,
            pltpu.SemaphoreType.DMA,
            pltpu.SemaphoreType.DMA,
        ],
    )
    def k(a0_hbm, a1_hbm, a2_hbm, combo_hbm, out_hbm,
          col_v, idx_v, rows_a, rows_b, sga, sgb, ssa, ssb):
        wid = lax.axis_index("s") * _NC + lax.axis_index("c")
        base = wid * per_w
        rep_base = wid * _KREP * n_combo

        ngrp = per_w // _LANES

        pltpu.sync_copy(a0_hbm.at[pl.ds(base, per_w)], col_v)

        def f0(i, c):
            s = pl.ds(i * _LANES, _LANES)
            rep = (lax.rem(i, _KREP) * n_combo) + rep_base
            idx_v[s] = col_v[s] * m0 + rep
            return c

        lax.fori_loop(0, ngrp, f0, 0)
        pltpu.sync_copy(a1_hbm.at[pl.ds(base, per_w)], col_v)

        def f1(i, c):
            s = pl.ds(i * _LANES, _LANES)
            idx_v[s] = idx_v[s] + col_v[s] * n2
            return c

        lax.fori_loop(0, ngrp, f1, 0)
        pltpu.sync_copy(a2_hbm.at[pl.ds(base, per_w)], col_v)

        def f2(i, c):
            s = pl.ds(i * _LANES, _LANES)
            idx_v[s] = idx_v[s] + col_v[s]
            return c

        lax.fori_loop(0, ngrp, f2, 0)

        rows = (rows_a, rows_b)
        sg = (sga, sgb)
        ss = (ssa, ssb)
        gath = {}
        stor = {}
        for g in range(nchunk):
            p = g % 2
            if g >= 2:
                stor[g - 2].wait()
            c = pltpu.make_async_copy(
                combo_hbm.at[idx_v.at[pl.ds(g * chunk, chunk)]],
                rows[p], sg[p])
            c.start()
            gath[g] = c
            if g >= 1:
                q = (g - 1) % 2
                gath[g - 1].wait()
                c = pltpu.make_async_copy(
                    rows[q], out_hbm.at[pl.ds(base + (g - 1) * chunk, chunk)],
                    ss[q])
                c.start()
                stor[g - 1] = c
        g = nchunk - 1
        gath[g].wait()
        c = pltpu.make_async_copy(
            rows[g % 2], out_hbm.at[pl.ds(base + g * chunk, chunk)],
            ss[g % 2])
        c.start()
        stor[g] = c
        stor[nchunk - 2].wait()
        stor[nchunk - 1].wait()

    return k(a0, a1, a2, combo)


def kernel(edge_attr, W0, W1, W2):
    ea = edge_attr.astype(jnp.int32)
    n1, n2 = W1.shape[0], W2.shape[0]
    combo = (W0[:, None, None, :] + W1[None, :, None, :]
             + W2[None, None, :, :]).reshape(-1, W0.shape[1])
    combo = jnp.tile(combo, (_NW * _KREP, 1))
    return _sc_combo_gather(ea[:, 0], ea[:, 1], ea[:, 2], combo,
                            n1=n1, n2=n2)


# speedup vs baseline: 9.1348x; 1.1398x over previous
import functools

import jax
import jax.numpy as jnp
from jax import lax
from jax.experimental import pallas as pl
from jax.experimental.pallas import tpu as pltpu
from jax.experimental.pallas import tpu_sc as plsc

_NC = 2
_NS = 16
_NW = _NC * _NS
_LANES = 16
_KREP = 16


def _pick_chunk(per_w: int) -> int:
    for c in range(min(per_w, 400), 7, -1):
        if c % 8 == 0 and per_w % c == 0:
            return c
    return 0


@functools.partial(jax.jit, static_argnames=("n1", "n2"))
def _sc_combo_gather(a0, a1, a2, combo, *, n1, n2):
    e = a0.shape[0]
    d = combo.shape[1]
    per_w = e // _NW
    chunk = _pick_chunk(per_w)
    assert per_w * _NW == e and chunk, f"unsupported edge count {e}"
    nchunk = per_w // chunk
    m0 = n1 * n2
    n_combo = combo.shape[0] // (_NW * _KREP)

    mesh = plsc.VectorSubcoreMesh(core_axis_name="c", subcore_axis_name="s")

    @functools.partial(
        pl.kernel,
        mesh=mesh,
        out_type=jax.ShapeDtypeStruct((e, d), jnp.float32),
        scratch_types=[
            pltpu.VMEM((per_w,), jnp.int32),
            pltpu.VMEM((per_w,), jnp.int32),
            pltpu.VMEM((chunk, d), jnp.float32),
            pltpu.VMEM((chunk, d), jnp.float32),
            pltpu.SemaphoreType.DMA,
            pltpu.SemaphoreType.DMA,
            pltpu.SemaphoreType.DMA,
            pltpu.SemaphoreType.DMA,
        ],
    )
    def k(a0_hbm, a1_hbm, a2_hbm, combo_hbm, out_hbm,
          col_v, idx_v, rows_a, rows_b, sga, sgb, ssa, ssb):
        wid = lax.axis_index("s") * _NC + lax.axis_index("c")
        base = wid * per_w
        rep_base = wid * _KREP * n_combo

        ngrp = per_w // _LANES

        pltpu.sync_copy(a0_hbm.at[pl.ds(base, per_w)], col_v)

        def f0(i, c):
            s = pl.ds(i * _LANES, _LANES)
            rep = (lax.rem(i, _KREP) * n_combo) + rep_base
            idx_v[s] = col_v[s] * m0 + rep
            return c

        lax.fori_loop(0, ngrp, f0, 0)
        pltpu.sync_copy(a1_hbm.at[pl.ds(base, per_w)], col_v)

        def f1(i, c):
            s = pl.ds(i * _LANES, _LANES)
            idx_v[s] = idx_v[s] + col_v[s] * n2
            return c

        lax.fori_loop(0, ngrp, f1, 0)
        pltpu.sync_copy(a2_hbm.at[pl.ds(base, per_w)], col_v)

        def f2(i, c):
            s = pl.ds(i * _LANES, _LANES)
            idx_v[s] = idx_v[s] + col_v[s]
            return c

        lax.fori_loop(0, ngrp, f2, 0)

        rows = (rows_a, rows_b)
        sg = (sga, sgb)
        ss = (ssa, ssb)
        gath = {}
        stor = {}
        for g in range(nchunk):
            p = g % 2
            if g >= 2:
                stor[g - 2].wait()
            c = pltpu.make_async_copy(
                combo_hbm.at[idx_v.at[pl.ds(g * chunk, chunk)]],
                rows[p], sg[p])
            c.start()
            gath[g] = c
            if g >= 1:
                q = (g - 1) % 2
                gath[g - 1].wait()
                c = pltpu.make_async_copy(
                    rows[q], out_hbm.at[pl.ds(base + (g - 1) * chunk, chunk)],
                    ss[q])
                c.start()
                stor[g - 1] = c
        g = nchunk - 1
        gath[g].wait()
        c = pltpu.make_async_copy(
            rows[g % 2], out_hbm.at[pl.ds(base + g * chunk, chunk)],
            ss[g % 2])
        c.start()
        stor[g] = c
        stor[nchunk - 2].wait()
        stor[nchunk - 1].wait()

    return k(a0, a1, a2, combo)


def kernel(edge_attr, W0, W1, W2):
    ea = edge_attr.astype(jnp.int32)
    n1, n2 = W1.shape[0], W2.shape[0]
    combo = (W0[:, None, None, :] + W1[None, :, None, :]
             + W2[None, None, :, :]).reshape(-1, W0.shape[1])
    combo = jnp.tile(combo, (_NW * _KREP, 1))
    return _sc_combo_gather(ea[:, 0], ea[:, 1], ea[:, 2], combo,
                            n1=n1, n2=n2)
